# Initial kernel scaffold; baseline (speedup 1.0000x reference)
import jax
import jax.numpy as jnp
from jax.experimental import pallas as pl


def kernel(x, pos_embedding):
    raise NotImplementedError("write your pallas kernel here")


# speedup vs baseline: 1.9199x; 1.9199x over previous
import jax
import jax.numpy as jnp
from jax.experimental import pallas as pl


_TT = 256


def _add_pos_kernel(x_ref, pos_ref, out_ref):
    pos = pos_ref[...]
    for b in range(x_ref.shape[0]):
        out_ref[b, :, :] = x_ref[b, :, :] + pos


def kernel(x, pos_embedding):
    B, T, D = x.shape
    tt = _TT if T % _TT == 0 else T
    grid = (T // tt,)
    return pl.pallas_call(
        _add_pos_kernel,
        grid=grid,
        in_specs=[
            pl.BlockSpec((B, tt, D), lambda i: (0, i, 0)),
            pl.BlockSpec((tt, D), lambda i: (i, 0)),
        ],
        out_specs=pl.BlockSpec((B, tt, D), lambda i: (0, i, 0)),
        out_shape=jax.ShapeDtypeStruct((B, T, D), x.dtype),
    )(x, pos_embedding)


# speedup vs baseline: 1.9610x; 1.0214x over previous
import jax
import jax.numpy as jnp
from jax.experimental import pallas as pl


_TT = 512


def _add_pos_kernel(x_ref, pos_ref, out_ref):
    pos = pos_ref[...]
    for b in range(x_ref.shape[0]):
        out_ref[b, :, :] = x_ref[b, :, :] + pos


def kernel(x, pos_embedding):
    B, T, D = x.shape
    tt = _TT if T % _TT == 0 else T
    grid = (T // tt,)
    return pl.pallas_call(
        _add_pos_kernel,
        grid=grid,
        in_specs=[
            pl.BlockSpec((B, tt, D), lambda i: (0, i, 0)),
            pl.BlockSpec((tt, D), lambda i: (i, 0)),
        ],
        out_specs=pl.BlockSpec((B, tt, D), lambda i: (0, i, 0)),
        out_shape=jax.ShapeDtypeStruct((B, T, D), x.dtype),
    )(x, pos_embedding)


# speedup vs baseline: 1.9907x; 1.0152x over previous
import jax
import jax.numpy as jnp
from jax.experimental import pallas as pl


_RT = 2048


def _add_pos_kernel(x_ref, pos_ref, out_ref):
    out_ref[0, :, :] = x_ref[0, :, :] + pos_ref[...]


def kernel(x, pos_embedding):
    B, T, D = x.shape
    rt = _RT if T % _RT == 0 else T
    grid = (T // rt, B)
    return pl.pallas_call(
        _add_pos_kernel,
        grid=grid,
        in_specs=[
            pl.BlockSpec((1, rt, D), lambda i, b: (b, i, 0)),
            pl.BlockSpec((rt, D), lambda i, b: (i, 0)),
        ],
        out_specs=pl.BlockSpec((1, rt, D), lambda i, b: (b, i, 0)),
        out_shape=jax.ShapeDtypeStruct((B, T, D), x.dtype),
    )(x, pos_embedding)
